# baseline (speedup 1.0000x reference)
import jax
import jax.numpy as jnp
from jax import lax
from jax.experimental import pallas as pl
from jax.experimental.pallas import tpu as pltpu
from jax.experimental.pallas import tpu_sc as plsc

B, L, H = 16, 1024, 768
HALF = L // 2
CHUNK = 32
NCHUNK = HALF // CHUNK
LANES = 16
NA = 26


def _cidx_body(x_ref, out_ref):
    xv = x_ref[...]
    mask = (xv != 0)
    tri = (lax.broadcasted_iota(jnp.int32, (L, L), 0)
           <= lax.broadcasted_iota(jnp.int32, (L, L), 1)).astype(jnp.float32)
    cs = jax.lax.dot_general(mask.astype(jnp.float32), tri,
                             (((1,), (0,)), ((), ())),
                             preferred_element_type=jnp.float32)
    positions = cs.astype(jnp.int32) * mask.astype(jnp.int32)
    inter = jnp.stack((xv, NA + positions), axis=-1)
    out_ref[...] = inter.reshape(B, 2 * L)


def _cidx(x):
    return pl.pallas_call(
        _cidx_body,
        out_shape=jax.ShapeDtypeStruct((B, 2 * L), jnp.int32),
    )(x)


def _sc_body(cidx_hbm, comb_hbm, gt_hbm, out_hbm,
             cidx, oidx, gidx, gtbuf, *bufsem):
    NSLOT = 2
    c = lax.axis_index("c")
    s = lax.axis_index("s")
    b = s
    half = c
    iota = lax.iota(jnp.int32, LANES)

    gbufs = bufsem[0:NSLOT]
    semg = bufsem[NSLOT:2 * NSLOT]
    semo = bufsem[2 * NSLOT:3 * NSLOT]
    semx = bufsem[3 * NSLOT]

    base = 1 + half * HALF
    sx = pltpu.async_copy(cidx_hbm.at[b, pl.ds(half * 2 * HALF, 2 * HALF)], cidx, semx)
    for k in range(NCHUNK):
        for j in range(CHUNK // LANES):
            oidx[k, pl.ds(j * LANES, LANES)] = base + k * CHUNK + j * LANES + iota
    sx.wait()

    gath = [None] * NSLOT
    scat = [None] * NSLOT

    def issue(kk):
        sl = kk % NSLOT
        if scat[sl] is not None:
            scat[sl].wait()
            scat[sl] = None
        gath[sl] = pltpu.async_copy(
            comb_hbm.at[cidx.at[pl.ds(kk * 2 * CHUNK, 2 * CHUNK)]],
            gbufs[sl], semg[sl])

    issue(0)
    for k in range(NCHUNK):
        slot = k % NSLOT
        if k + 1 < NCHUNK:
            issue(k + 1)
        gath[slot].wait()
        gb = gbufs[slot]

        def add_row(t, _, gb=gb):
            for j in range(H // LANES):
                sl = pl.ds(j * LANES, LANES)
                gb[t, sl] = gb[2 * t, sl] + gb[2 * t + 1, sl]
            return 0

        lax.fori_loop(0, CHUNK, add_row, 0)
        scat[slot] = pltpu.async_copy(
            gb.at[pl.ds(0, CHUNK)], out_hbm.at[b].at[oidx.at[k]], semo[slot])
    for sl in range(NSLOT):
        if scat[sl] is not None:
            scat[sl].wait()

    gidx[pl.ds(0, LANES)] = iota * 0
    pltpu.async_copy(gt_hbm.at[gidx], gtbuf, semx).wait()
    pltpu.async_copy(gtbuf, out_hbm.at[b].at[gidx], semx).wait()


def kernel(x, atom_table, pos_table, graph_token):
    cidx = _cidx(x)
    comb = jnp.concatenate([atom_table, pos_table], axis=0)
    mesh = plsc.VectorSubcoreMesh(
        core_axis_name="c", subcore_axis_name="s", num_cores=2, num_subcores=16)
    f = pl.kernel(
        _sc_body,
        out_type=jax.ShapeDtypeStruct((B, L + 1, H), jnp.float32),
        mesh=mesh,
        scratch_types=[
            pltpu.VMEM((2 * HALF,), jnp.int32),
            pltpu.VMEM((NCHUNK, CHUNK), jnp.int32),
            pltpu.VMEM((LANES,), jnp.int32),
            pltpu.VMEM((LANES, H), jnp.float32),
            pltpu.VMEM((2 * CHUNK, H), jnp.float32),
            pltpu.VMEM((2 * CHUNK, H), jnp.float32),
            pltpu.SemaphoreType.DMA,
            pltpu.SemaphoreType.DMA,
            pltpu.SemaphoreType.DMA,
            pltpu.SemaphoreType.DMA,
            pltpu.SemaphoreType.DMA,
        ],
    )
    return f(cidx, comb, graph_token, )


# speedup vs baseline: 1.2152x; 1.2152x over previous
import jax
import jax.numpy as jnp
from jax import lax
from jax.experimental import pallas as pl
from jax.experimental.pallas import tpu as pltpu
from jax.experimental.pallas import tpu_sc as plsc

B, L, H = 16, 1024, 768
HALF = L // 2
CHUNK = 32
NCHUNK = HALF // CHUNK
LANES = 16


def _positions_body(x_ref, out_ref):
    mask = (x_ref[...] != 0)
    tri = (lax.broadcasted_iota(jnp.int32, (L, L), 0)
           <= lax.broadcasted_iota(jnp.int32, (L, L), 1)).astype(jnp.float32)
    cs = jax.lax.dot_general(mask.astype(jnp.float32), tri,
                             (((1,), (0,)), ((), ())),
                             preferred_element_type=jnp.float32)
    out_ref[...] = cs.astype(jnp.int32) * mask.astype(jnp.int32)


def _positions(x):
    return pl.pallas_call(
        _positions_body,
        out_shape=jax.ShapeDtypeStruct((B, L), jnp.int32),
    )(x)


def _sc_body(x_hbm, posn_hbm, atom_hbm, pos_hbm, gt_hbm, out_hbm,
             aidx, pidx, oidx, gidx,
             abuf0, abuf1, pbuf0, pbuf1, gtbuf,
             sema0, sema1, semp0, semp1, semo0, semo1):
    c = lax.axis_index("c")
    s = lax.axis_index("s")
    b = s
    half = c
    iota = lax.iota(jnp.int32, LANES)

    abufs = (abuf0, abuf1)
    pbufs = (pbuf0, pbuf1)
    semas = (sema0, sema1)
    semps = (semp0, semp1)
    semos = (semo0, semo1)

    base = 1 + half * HALF
    sa = pltpu.async_copy(x_hbm.at[b, pl.ds(half * HALF, HALF)], aidx, sema0)
    sp = pltpu.async_copy(posn_hbm.at[b, pl.ds(half * HALF, HALF)], pidx, semp0)
    for k in range(NCHUNK):
        for j in range(CHUNK // LANES):
            oidx[k, pl.ds(j * LANES, LANES)] = base + k * CHUNK + j * LANES + iota
    sa.wait()
    sp.wait()

    gath = [None, None]
    scat = [None, None]
    gath[0] = (pltpu.async_copy(atom_hbm.at[aidx.at[pl.ds(0, CHUNK)]], abufs[0], semas[0]),
               pltpu.async_copy(pos_hbm.at[pidx.at[pl.ds(0, CHUNK)]], pbufs[0], semps[0]))
    for k in range(NCHUNK):
        slot = k % 2
        nslot = (k + 1) % 2
        if k + 1 < NCHUNK:
            if scat[nslot] is not None:
                scat[nslot].wait()
                scat[nslot] = None
            gath[nslot] = (
                pltpu.async_copy(atom_hbm.at[aidx.at[pl.ds((k + 1) * CHUNK, CHUNK)]],
                                 abufs[nslot], semas[nslot]),
                pltpu.async_copy(pos_hbm.at[pidx.at[pl.ds((k + 1) * CHUNK, CHUNK)]],
                                 pbufs[nslot], semps[nslot]))
        ga, gp = gath[slot]
        ga.wait()
        gp.wait()
        ab, pb = abufs[slot], pbufs[slot]

        def add_row(t, _, ab=ab, pb=pb):
            for j in range(H // LANES):
                sl = pl.ds(j * LANES, LANES)
                ab[t, sl] = ab[t, sl] + pb[t, sl]
            return 0

        lax.fori_loop(0, CHUNK, add_row, 0)
        scat[slot] = pltpu.async_copy(ab, out_hbm.at[b].at[oidx.at[k]], semos[slot])
    for sl in (0, 1):
        if scat[sl] is not None:
            scat[sl].wait()

    gidx[pl.ds(0, LANES)] = iota * 0
    pltpu.async_copy(gt_hbm.at[gidx], gtbuf, semo0).wait()
    pltpu.async_copy(gtbuf, out_hbm.at[b].at[gidx], semo0).wait()


def kernel(x, atom_table, pos_table, graph_token):
    positions = _positions(x)
    mesh = plsc.VectorSubcoreMesh(
        core_axis_name="c", subcore_axis_name="s", num_cores=2, num_subcores=16)
    f = pl.kernel(
        _sc_body,
        out_type=jax.ShapeDtypeStruct((B, L + 1, H), jnp.float32),
        mesh=mesh,
        scratch_types=[
            pltpu.VMEM((HALF,), jnp.int32),
            pltpu.VMEM((HALF,), jnp.int32),
            pltpu.VMEM((NCHUNK, CHUNK), jnp.int32),
            pltpu.VMEM((LANES,), jnp.int32),
            pltpu.VMEM((CHUNK, H), jnp.float32),
            pltpu.VMEM((CHUNK, H), jnp.float32),
            pltpu.VMEM((CHUNK, H), jnp.float32),
            pltpu.VMEM((CHUNK, H), jnp.float32),
            pltpu.VMEM((LANES, H), jnp.float32),
            pltpu.SemaphoreType.DMA,
            pltpu.SemaphoreType.DMA,
            pltpu.SemaphoreType.DMA,
            pltpu.SemaphoreType.DMA,
            pltpu.SemaphoreType.DMA,
            pltpu.SemaphoreType.DMA,
        ],
    )
    return f(x, positions, atom_table, pos_table, graph_token)
